# baseline (speedup 1.0000x reference)
import jax
import jax.numpy as jnp
from jax.experimental import pallas as pl


def _copy_body(e_ref, eo_ref):
    eo_ref[...] = e_ref[...]


def kernel(nodes, edge_index, edges=None, u=None, batch=None):
    if batch is None:
        batch = jnp.zeros((nodes.shape[0],), dtype=jnp.int32)

    n_edges, d_edge = edges.shape
    e2 = edges.reshape(n_edges * d_edge // 128, 128)
    g = 5
    eb = e2.shape[0] // g
    e_o = pl.pallas_call(
        _copy_body,
        grid=(g,),
        in_specs=[pl.BlockSpec((eb, 128), lambda i: (i, 0))],
        out_specs=pl.BlockSpec((eb, 128), lambda i: (i, 0)),
        out_shape=jax.ShapeDtypeStruct(e2.shape, e2.dtype),
    )(e2)
    return (nodes, edge_index, e_o.reshape(edges.shape), u, batch)


# speedup vs baseline: 1.0987x; 1.0987x over previous
import jax
import jax.numpy as jnp
from jax.experimental import pallas as pl


def _copy_body(e_ref, eo_ref):
    eo_ref[...] = e_ref[...]


def kernel(nodes, edge_index, edges=None, u=None, batch=None):
    if batch is None:
        batch = jnp.zeros((nodes.shape[0],), dtype=jnp.int32)

    n_edges, d_edge = edges.shape
    g = 20
    eb = n_edges // g
    e_o = pl.pallas_call(
        _copy_body,
        grid=(g,),
        in_specs=[pl.BlockSpec((eb, d_edge), lambda i: (i, 0))],
        out_specs=pl.BlockSpec((eb, d_edge), lambda i: (i, 0)),
        out_shape=jax.ShapeDtypeStruct(edges.shape, edges.dtype),
    )(edges)
    return (nodes, edge_index, e_o, u, batch)
